# Initial kernel scaffold; baseline (speedup 1.0000x reference)
import jax
import jax.numpy as jnp
from jax.experimental import pallas as pl


def kernel(positions, atomic_subsystem_indices):
    raise NotImplementedError("write your pallas kernel here")


# speedup vs baseline: 8.5245x; 8.5245x over previous
import functools

import jax
import jax.numpy as jnp
from jax import lax
from jax.experimental import pallas as pl
from jax.experimental.pallas import tpu as pltpu
from jax.experimental.pallas import tpu_sc as plsc

N_SYS = 1000
AP = 50
NPAIR = AP * (AP - 1)
NCHUNK = (NPAIR + 15) // 16
GSYS = 4
NG = N_SYS // GSYS
POSW = AP * 3

_L = 16


def _sc_pair_kernel(pos_hbm, asi_hbm, r_hbm, d_hbm, pi_hbm,
                    pos_v, asi_v, r_v, d_v, pii_v, pij_v):
    info = plsc.get_sparse_core_info()
    nc, ns = info.num_cores, info.num_subcores
    nw = nc * ns
    w = lax.axis_index("s") * nc + lax.axis_index("c")
    base, rem = NG // nw, NG % nw
    lo = w * base + jnp.minimum(w, rem)
    cnt = base + jnp.where(w < rem, 1, 0)

    iota = lax.iota(jnp.int32, _L)
    fone5 = jnp.full((_L,), 1.5, jnp.float32)
    magic = jnp.full((_L,), 0x5F3759DF, jnp.int32)

    def group_body(gi, _):
        g = lo + gi
        pltpu.sync_copy(pos_hbm.at[g], pos_v)
        pltpu.sync_copy(asi_hbm.at[g], asi_v)

        def sys_body(t, _):
            pbase = t * POSW
            obase_d = t * NPAIR
            obase_r = t * (3 * NPAIR)
            start = plsc.load_gather(asi_v, [iota * 0 + t * AP]) * AP

            def chunk_body(n, _):
                ks = jnp.minimum(n * _L, NPAIR - _L)
                k = ks + iota
                li = (k * 1338) >> 16
                rr = k - li * 49
                lj = jnp.where(rr < li, rr, rr + 1)
                i3 = pbase + li * 3
                j3 = pbase + lj * 3
                k3 = obase_r + k * 3
                sq = jnp.zeros((_L,), jnp.float32)
                for c in range(3):
                    gci = plsc.load_gather(pos_v, [i3 + c])
                    gcj = plsc.load_gather(pos_v, [j3 + c])
                    rc = gcj - gci
                    plsc.store_scatter(r_v, [k3 + c], rc)
                    sq = sq + rc * rc
                y = plsc.bitcast(magic - (plsc.bitcast(sq, jnp.int32) >> 1),
                                 jnp.float32)
                h = sq * 0.5
                y = y * (fone5 - h * y * y)
                y = y * (fone5 - h * y * y)
                y = y * (fone5 - h * y * y)
                d_v[pl.ds(obase_d + ks, _L)] = sq * y
                pii_v[pl.ds(obase_d + ks, _L)] = start + li
                pij_v[pl.ds(obase_d + ks, _L)] = start + lj
                return 0

            lax.fori_loop(0, NCHUNK, chunk_body, 0)
            return 0

        lax.fori_loop(0, GSYS, sys_body, 0)
        pltpu.sync_copy(r_v, r_hbm.at[g])
        pltpu.sync_copy(d_v, d_hbm.at[g])
        pltpu.sync_copy(pii_v, pi_hbm.at[g])
        pltpu.sync_copy(pij_v, pi_hbm.at[NG + g])
        return 0

    lax.fori_loop(0, cnt, group_body, 0)


@functools.partial(
    pl.kernel,
    out_type=(
        jax.ShapeDtypeStruct((NG, GSYS * 3 * NPAIR), jnp.float32),
        jax.ShapeDtypeStruct((NG, GSYS * NPAIR), jnp.float32),
        jax.ShapeDtypeStruct((2 * NG, GSYS * NPAIR), jnp.int32),
    ),
    mesh=plsc.VectorSubcoreMesh(core_axis_name="c", subcore_axis_name="s"),
    compiler_params=pltpu.CompilerParams(needs_layout_passes=False),
    scratch_types=[
        pltpu.VMEM((GSYS * POSW,), jnp.float32),
        pltpu.VMEM((GSYS * AP,), jnp.int32),
        pltpu.VMEM((GSYS * 3 * NPAIR,), jnp.float32),
        pltpu.VMEM((GSYS * NPAIR,), jnp.float32),
        pltpu.VMEM((GSYS * NPAIR,), jnp.int32),
        pltpu.VMEM((GSYS * NPAIR,), jnp.int32),
    ],
)
def _pair_list_sc(pos_hbm, asi_hbm, r_hbm, d_hbm, pi_hbm, *scratch):
    _sc_pair_kernel(pos_hbm, asi_hbm, r_hbm, d_hbm, pi_hbm, *scratch)


def kernel(positions, atomic_subsystem_indices):
    pos2 = positions.reshape(NG, GSYS * POSW)
    asi2 = atomic_subsystem_indices.reshape(NG, GSYS * AP)
    r2, d2, pi2 = _pair_list_sc(pos2, asi2)
    pair_indices = pi2.reshape(2, N_SYS * NPAIR)
    d_ij = d2.reshape(N_SYS * NPAIR, 1)
    r_ij = r2.reshape(N_SYS * NPAIR, 3)
    return (pair_indices, d_ij, r_ij)


# speedup vs baseline: 9.4123x; 1.1041x over previous
import functools

import jax
import jax.numpy as jnp
from jax import lax
from jax.experimental import pallas as pl
from jax.experimental.pallas import tpu as pltpu
from jax.experimental.pallas import tpu_sc as plsc

N_SYS = 1000
AP = 50
NPAIR = AP * (AP - 1)
NCHUNK = (NPAIR + 15) // 16
GSYS = 4
NG = N_SYS // GSYS
POSW = AP * 3

_L = 16


def _sc_pair_kernel(pos_hbm, asi_hbm, r_hbm, d_hbm, pi_hbm,
                    pos_v, asi_v, r_v, d_v, pii_v, pij_v,
                    li_t, lj_t, li3_t, lj3_t):
    info = plsc.get_sparse_core_info()
    nc, ns = info.num_cores, info.num_subcores
    nw = nc * ns
    w = lax.axis_index("s") * nc + lax.axis_index("c")
    base, rem = NG // nw, NG % nw
    lo = w * base + jnp.minimum(w, rem)
    cnt = base + jnp.where(w < rem, 1, 0)

    iota = lax.iota(jnp.int32, _L)
    iota3 = iota * 3
    fone5 = jnp.full((_L,), 1.5, jnp.float32)
    magic = jnp.full((_L,), 0x5F3759DF, jnp.int32)

    def table_body(n, _):
        k = jnp.minimum(n * _L, NPAIR - _L) + iota
        li = (k * 1338) >> 16
        rr = k - li * 49
        lj = jnp.where(rr < li, rr, rr + 1)
        tb = n * _L
        li_t[pl.ds(tb, _L)] = li
        lj_t[pl.ds(tb, _L)] = lj
        li3_t[pl.ds(tb, _L)] = li * 3
        lj3_t[pl.ds(tb, _L)] = lj * 3
        return 0

    lax.fori_loop(0, NCHUNK, table_body, 0)

    def group_body(gi, _):
        g = lo + gi
        pltpu.sync_copy(pos_hbm.at[g], pos_v)
        pltpu.sync_copy(asi_hbm.at[g], asi_v)

        def sys_body(t, _):
            pbase = t * POSW
            obase_d = t * NPAIR
            obase_r = t * (3 * NPAIR)
            start = plsc.load_gather(asi_v, [iota * 0 + t * AP]) * AP

            @plsc.parallel_loop(0, NCHUNK, unroll=2)
            def chunk_body(n):
                ks = jnp.minimum(n * _L, NPAIR - _L)
                tb = n * _L
                li = li_t[pl.ds(tb, _L)]
                lj = lj_t[pl.ds(tb, _L)]
                i3 = li3_t[pl.ds(tb, _L)] + pbase
                j3 = lj3_t[pl.ds(tb, _L)] + pbase
                k3 = (obase_r + ks * 3) + iota3
                sq = jnp.zeros((_L,), jnp.float32)
                for c in range(3):
                    gci = plsc.load_gather(pos_v, [i3 + c if c else i3])
                    gcj = plsc.load_gather(pos_v, [j3 + c if c else j3])
                    rc = gcj - gci
                    plsc.store_scatter(r_v, [k3 + c if c else k3], rc)
                    sq = sq + rc * rc
                y = plsc.bitcast(magic - (plsc.bitcast(sq, jnp.int32) >> 1),
                                 jnp.float32)
                h = sq * 0.5
                y = y * (fone5 - h * y * y)
                y = y * (fone5 - h * y * y)
                d_v[pl.ds(obase_d + ks, _L)] = sq * y
                pii_v[pl.ds(obase_d + ks, _L)] = start + li
                pij_v[pl.ds(obase_d + ks, _L)] = start + lj

            return 0

        lax.fori_loop(0, GSYS, sys_body, 0)
        pltpu.sync_copy(r_v, r_hbm.at[g])
        pltpu.sync_copy(d_v, d_hbm.at[g])
        pltpu.sync_copy(pii_v, pi_hbm.at[g])
        pltpu.sync_copy(pij_v, pi_hbm.at[NG + g])
        return 0

    lax.fori_loop(0, cnt, group_body, 0)


@functools.partial(
    pl.kernel,
    out_type=(
        jax.ShapeDtypeStruct((NG, GSYS * 3 * NPAIR), jnp.float32),
        jax.ShapeDtypeStruct((NG, GSYS * NPAIR), jnp.float32),
        jax.ShapeDtypeStruct((2 * NG, GSYS * NPAIR), jnp.int32),
    ),
    mesh=plsc.VectorSubcoreMesh(core_axis_name="c", subcore_axis_name="s"),
    compiler_params=pltpu.CompilerParams(needs_layout_passes=False),
    scratch_types=[
        pltpu.VMEM((GSYS * POSW,), jnp.float32),
        pltpu.VMEM((GSYS * AP,), jnp.int32),
        pltpu.VMEM((GSYS * 3 * NPAIR,), jnp.float32),
        pltpu.VMEM((GSYS * NPAIR,), jnp.float32),
        pltpu.VMEM((GSYS * NPAIR,), jnp.int32),
        pltpu.VMEM((GSYS * NPAIR,), jnp.int32),
        pltpu.VMEM((NCHUNK * _L,), jnp.int32),
        pltpu.VMEM((NCHUNK * _L,), jnp.int32),
        pltpu.VMEM((NCHUNK * _L,), jnp.int32),
        pltpu.VMEM((NCHUNK * _L,), jnp.int32),
    ],
)
def _pair_list_sc(pos_hbm, asi_hbm, r_hbm, d_hbm, pi_hbm, *scratch):
    _sc_pair_kernel(pos_hbm, asi_hbm, r_hbm, d_hbm, pi_hbm, *scratch)


def kernel(positions, atomic_subsystem_indices):
    pos2 = positions.reshape(NG, GSYS * POSW)
    asi2 = atomic_subsystem_indices.reshape(NG, GSYS * AP)
    r2, d2, pi2 = _pair_list_sc(pos2, asi2)
    pair_indices = pi2.reshape(2, N_SYS * NPAIR)
    d_ij = d2.reshape(N_SYS * NPAIR, 1)
    r_ij = r2.reshape(N_SYS * NPAIR, 3)
    return (pair_indices, d_ij, r_ij)


# speedup vs baseline: 9.5803x; 1.0178x over previous
import functools

import jax
import jax.numpy as jnp
from jax import lax
from jax.experimental import pallas as pl
from jax.experimental.pallas import tpu as pltpu
from jax.experimental.pallas import tpu_sc as plsc

N_SYS = 1000
AP = 50
NPAIR = AP * (AP - 1)
NCHUNK = (NPAIR + 15) // 16
GSYS = 4
NG = N_SYS // GSYS
POSW = AP * 3

_L = 16


def _sc_pair_kernel(pos_hbm, asi_hbm, r_hbm, d_hbm, pi_hbm,
                    pos_v0, pos_v1, asi_v0, asi_v1, r_v0, r_v1,
                    d_v0, d_v1, pii_v0, pii_v1, pij_v0, pij_v1,
                    li_t, lj_t, li3_t, lj3_t,
                    in_sem0, in_sem1, out_sem0, out_sem1):
    pos_v = (pos_v0, pos_v1)
    asi_v = (asi_v0, asi_v1)
    r_v = (r_v0, r_v1)
    d_v = (d_v0, d_v1)
    pii_v = (pii_v0, pii_v1)
    pij_v = (pij_v0, pij_v1)
    in_sems = (in_sem0, in_sem1)
    out_sems = (out_sem0, out_sem1)
    info = plsc.get_sparse_core_info()
    nc, ns = info.num_cores, info.num_subcores
    nw = nc * ns
    w = lax.axis_index("s") * nc + lax.axis_index("c")
    base, rem = NG // nw, NG % nw
    lo = w * base + jnp.minimum(w, rem)
    cnt = base + jnp.where(w < rem, 1, 0)

    iota = lax.iota(jnp.int32, _L)
    iota3 = iota * 3
    fone5 = jnp.full((_L,), 1.5, jnp.float32)
    magic = jnp.full((_L,), 0x5F3759DF, jnp.int32)

    def table_body(n, _):
        k = jnp.minimum(n * _L, NPAIR - _L) + iota
        li = (k * 1338) >> 16
        rr = k - li * 49
        lj = jnp.where(rr < li, rr, rr + 1)
        tb = n * _L
        li_t[pl.ds(tb, _L)] = li
        lj_t[pl.ds(tb, _L)] = lj
        li3_t[pl.ds(tb, _L)] = li * 3
        lj3_t[pl.ds(tb, _L)] = lj * 3
        return 0

    lax.fori_loop(0, NCHUNK, table_body, 0)

    def compute_group(pos_b, asi_b, r_b, d_b, pii_b, pij_b):
        def sys_body(t, _):
            pbase = t * POSW
            obase_d = t * NPAIR
            obase_r = t * (3 * NPAIR)
            start = plsc.load_gather(asi_b, [iota * 0 + t * AP]) * AP

            @plsc.parallel_loop(0, NCHUNK, unroll=2)
            def chunk_body(n):
                ks = jnp.minimum(n * _L, NPAIR - _L)
                tb = n * _L
                li = li_t[pl.ds(tb, _L)]
                lj = lj_t[pl.ds(tb, _L)]
                i3 = li3_t[pl.ds(tb, _L)] + pbase
                j3 = lj3_t[pl.ds(tb, _L)] + pbase
                k3 = (obase_r + ks * 3) + iota3
                sq = jnp.zeros((_L,), jnp.float32)
                for c in range(3):
                    gci = plsc.load_gather(pos_b, [i3 + c if c else i3])
                    gcj = plsc.load_gather(pos_b, [j3 + c if c else j3])
                    rc = gcj - gci
                    plsc.store_scatter(r_b, [k3 + c if c else k3], rc)
                    sq = sq + rc * rc
                y = plsc.bitcast(magic - (plsc.bitcast(sq, jnp.int32) >> 1),
                                 jnp.float32)
                h = sq * 0.5
                y = y * (fone5 - h * y * y)
                y = y * (fone5 - h * y * y)
                d_b[pl.ds(obase_d + ks, _L)] = sq * y
                pii_b[pl.ds(obase_d + ks, _L)] = start + li
                pij_b[pl.ds(obase_d + ks, _L)] = start + lj

            return 0

        lax.fori_loop(0, GSYS, sys_body, 0)

    for j in (0, 1):
        @pl.when(j < cnt)
        def _(j=j):
            pltpu.async_copy(pos_hbm.at[lo + j], pos_v[j], in_sems[j])
            pltpu.async_copy(asi_hbm.at[lo + j], asi_v[j], in_sems[j])

    def step(t, _):
        for j in (0, 1):
            gi = 2 * t + j

            @pl.when(gi < cnt)
            def _(j=j, gi=gi):
                g = lo + gi
                pltpu.make_async_copy(
                    pos_hbm.at[0], pos_v[j], in_sems[j]).wait()
                pltpu.make_async_copy(
                    asi_hbm.at[0], asi_v[j], in_sems[j]).wait()

                @pl.when(t > 0)
                def _():
                    pltpu.make_async_copy(
                        r_v[j], r_hbm.at[0], out_sems[j]).wait()
                    pltpu.make_async_copy(
                        d_v[j], d_hbm.at[0], out_sems[j]).wait()
                    pltpu.make_async_copy(
                        pii_v[j], pi_hbm.at[0], out_sems[j]).wait()
                    pltpu.make_async_copy(
                        pij_v[j], pi_hbm.at[0], out_sems[j]).wait()

                compute_group(pos_v[j], asi_v[j], r_v[j],
                              d_v[j], pii_v[j], pij_v[j])

                @pl.when(gi + 2 < cnt)
                def _():
                    pltpu.async_copy(
                        pos_hbm.at[g + 2], pos_v[j], in_sems[j])
                    pltpu.async_copy(
                        asi_hbm.at[g + 2], asi_v[j], in_sems[j])

                pltpu.async_copy(r_v[j], r_hbm.at[g], out_sems[j])
                pltpu.async_copy(d_v[j], d_hbm.at[g], out_sems[j])
                pltpu.async_copy(pii_v[j], pi_hbm.at[g], out_sems[j])
                pltpu.async_copy(pij_v[j], pi_hbm.at[NG + g], out_sems[j])

        return 0

    lax.fori_loop(0, (cnt + 1) // 2, step, 0)

    for j in (0, 1):
        @pl.when(j < cnt)
        def _(j=j):
            pltpu.make_async_copy(r_v[j], r_hbm.at[0], out_sems[j]).wait()
            pltpu.make_async_copy(d_v[j], d_hbm.at[0], out_sems[j]).wait()
            pltpu.make_async_copy(pii_v[j], pi_hbm.at[0],
                                  out_sems[j]).wait()
            pltpu.make_async_copy(pij_v[j], pi_hbm.at[0],
                                  out_sems[j]).wait()


@functools.partial(
    pl.kernel,
    out_type=(
        jax.ShapeDtypeStruct((NG, GSYS * 3 * NPAIR), jnp.float32),
        jax.ShapeDtypeStruct((NG, GSYS * NPAIR), jnp.float32),
        jax.ShapeDtypeStruct((2 * NG, GSYS * NPAIR), jnp.int32),
    ),
    mesh=plsc.VectorSubcoreMesh(core_axis_name="c", subcore_axis_name="s"),
    compiler_params=pltpu.CompilerParams(needs_layout_passes=False),
    scratch_types=[
        pltpu.VMEM((GSYS * POSW,), jnp.float32),
        pltpu.VMEM((GSYS * POSW,), jnp.float32),
        pltpu.VMEM((GSYS * AP,), jnp.int32),
        pltpu.VMEM((GSYS * AP,), jnp.int32),
        pltpu.VMEM((GSYS * 3 * NPAIR,), jnp.float32),
        pltpu.VMEM((GSYS * 3 * NPAIR,), jnp.float32),
        pltpu.VMEM((GSYS * NPAIR,), jnp.float32),
        pltpu.VMEM((GSYS * NPAIR,), jnp.float32),
        pltpu.VMEM((GSYS * NPAIR,), jnp.int32),
        pltpu.VMEM((GSYS * NPAIR,), jnp.int32),
        pltpu.VMEM((GSYS * NPAIR,), jnp.int32),
        pltpu.VMEM((GSYS * NPAIR,), jnp.int32),
        pltpu.VMEM((NCHUNK * _L,), jnp.int32),
        pltpu.VMEM((NCHUNK * _L,), jnp.int32),
        pltpu.VMEM((NCHUNK * _L,), jnp.int32),
        pltpu.VMEM((NCHUNK * _L,), jnp.int32),
        pltpu.SemaphoreType.DMA,
        pltpu.SemaphoreType.DMA,
        pltpu.SemaphoreType.DMA,
        pltpu.SemaphoreType.DMA,
    ],
)
def _pair_list_sc(pos_hbm, asi_hbm, r_hbm, d_hbm, pi_hbm, *scratch):
    _sc_pair_kernel(pos_hbm, asi_hbm, r_hbm, d_hbm, pi_hbm, *scratch)


def kernel(positions, atomic_subsystem_indices):
    pos2 = positions.reshape(NG, GSYS * POSW)
    asi2 = atomic_subsystem_indices.reshape(NG, GSYS * AP)
    r2, d2, pi2 = _pair_list_sc(pos2, asi2)
    pair_indices = pi2.reshape(2, N_SYS * NPAIR)
    d_ij = d2.reshape(N_SYS * NPAIR, 1)
    r_ij = r2.reshape(N_SYS * NPAIR, 3)
    return (pair_indices, d_ij, r_ij)
